# Initial kernel scaffold; baseline (speedup 1.0000x reference)
import jax
import jax.numpy as jnp
from jax.experimental import pallas as pl


def kernel(x, table, gamma, beta):
    raise NotImplementedError("write your pallas kernel here")


# speedup vs baseline: 2.1858x; 2.1858x over previous
import functools

import jax
import jax.numpy as jnp
from jax import lax
from jax.experimental import pallas as pl
from jax.experimental.pallas import tpu as pltpu
from jax.experimental.pallas import tpu_sc as plsc

N_GENES = 100000
D = 64
B = 4096
L = 200
EPS = 1e-5

BL = B * L
NC = 2
NS = 16
NW = NC * NS
PW = BL // NW
CH = 128
NCHUNK = PW // CH


def _rsqrt_newton(x):
    i = lax.bitcast_convert_type(x, jnp.int32)
    i = jnp.int32(0x5F3759DF) - (i >> 1)
    y = lax.bitcast_convert_type(i, jnp.float32)
    for _ in range(3):
        y = y * (jnp.float32(1.5) - jnp.float32(0.5) * x * y * y)
    return y


def _sc_body(x_hbm, table_hbm, gamma_hbm, beta_hbm, out_hbm,
             idx_v, rows_v, g_v, b_v, sem):
    wid = lax.axis_index("s") * NC + lax.axis_index("c")

    pltpu.sync_copy(x_hbm.at[wid], idx_v)
    pltpu.sync_copy(gamma_hbm, g_v)
    pltpu.sync_copy(beta_hbm, b_v)
    g = [g_v[pl.ds(16 * k, 16)] for k in range(4)]
    bta = [b_v[pl.ds(16 * k, 16)] for k in range(4)]

    def chunk_body(c, carry):
        pltpu.async_copy(table_hbm.at[idx_v.at[c]], rows_v, sem).wait()

        def row_body(r, rcarry):
            v = [rows_v[r, pl.ds(16 * k, 16)] for k in range(4)]
            s = (v[0] + v[1]) + (v[2] + v[3])
            s2 = (v[0] * v[0] + v[1] * v[1]) + (v[2] * v[2] + v[3] * v[3])
            rsum = jnp.sum(s)
            rsq = jnp.sum(s2)
            mean = rsum * jnp.float32(1.0 / D)
            var = rsq * jnp.float32(1.0 / D) - mean * mean
            rstd = _rsqrt_newton(var + jnp.float32(EPS))
            for k in range(4):
                rows_v[r, pl.ds(16 * k, 16)] = (v[k] - mean) * rstd * g[k] + bta[k]
            return rcarry

        lax.fori_loop(0, CH, row_body, 0, unroll=2)

        base = pl.multiple_of((wid * NCHUNK + c) * CH, CH)
        pltpu.sync_copy(rows_v, out_hbm.at[pl.ds(base, CH)])
        return carry

    lax.fori_loop(0, NCHUNK, chunk_body, 0)


@jax.jit
def kernel(x, table, gamma, beta):
    xw = x.astype(jnp.int32).reshape(NW, NCHUNK, CH)
    mesh = plsc.VectorSubcoreMesh(core_axis_name="c", subcore_axis_name="s")
    run = functools.partial(
        pl.kernel,
        mesh=mesh,
        out_type=jax.ShapeDtypeStruct((BL, D), jnp.float32),
        scratch_types=[
            pltpu.VMEM((NCHUNK, CH), jnp.int32),
            pltpu.VMEM((CH, D), jnp.float32),
            pltpu.VMEM((D,), jnp.float32),
            pltpu.VMEM((D,), jnp.float32),
            pltpu.SemaphoreType.DMA,
        ],
        compiler_params=pltpu.CompilerParams(
            needs_layout_passes=False, use_tc_tiling_on_sc=False),
    )(_sc_body)
    out = run(xw, table, gamma, beta)
    return out.reshape(B, L, D)
